# baseline (speedup 1.0000x reference)
import functools

import jax
import jax.numpy as jnp
from jax import lax
from jax.experimental import pallas as pl
from jax.experimental.pallas import tpu as pltpu
from jax.experimental.pallas import tpu_sc as plsc

_N = 100000
_H = 64
_TXN_IN = 256

_CHUNK = 1024
_SUB = 128
_NCHUNKS = (_N + _CHUNK - 1) // _CHUNK
_LASTBASE = _N - _CHUNK
_NW = 32

_sc_mesh = plsc.VectorSubcoreMesh(core_axis_name="c", subcore_axis_name="s")


def _gather_tables(idx_refs, mem_refs, out_refs, idx_v, rows_v, sem, wid):
    for idx_hbm, mem_hbm, out_hbm in zip(idx_refs, mem_refs, out_refs):
        for j in range((_NCHUNKS + _NW - 1) // _NW):
            c = wid + _NW * j

            @pl.when(c < _NCHUNKS)
            def _():
                base = jnp.minimum(c * _CHUNK, _LASTBASE)
                pltpu.sync_copy(idx_hbm.at[pl.ds(base, _CHUNK)], idx_v)
                cps = [
                    pltpu.async_copy(
                        mem_hbm.at[idx_v.at[pl.ds(q * _SUB, _SUB)]],
                        rows_v.at[pl.ds(q * _SUB, _SUB)],
                        sem,
                    )
                    for q in range(8)
                ]
                for cp in cps:
                    cp.wait()
                pltpu.sync_copy(rows_v, out_hbm.at[pl.ds(base, _CHUNK)])


_sc_scratch = [
    pltpu.VMEM((_CHUNK,), jnp.int32),
    pltpu.VMEM((_CHUNK, _H), jnp.float32),
    pltpu.SemaphoreType.DMA,
]
_sc_params = pltpu.CompilerParams(use_tc_tiling_on_sc=False)


@functools.partial(
    pl.kernel, mesh=_sc_mesh,
    out_type=[jax.ShapeDtypeStruct((_N, _H), jnp.float32)] * 2,
    scratch_types=_sc_scratch, compiler_params=_sc_params,
)
def _gather_small(idx_a, idx_e, mem_a, mem_e, out_a, out_e,
                  idx_v, rows_v, sem):
    wid = lax.axis_index("s") * 2 + lax.axis_index("c")
    _gather_tables((idx_a, idx_e), (mem_a, mem_e), (out_a, out_e),
                   idx_v, rows_v, sem, wid)


@functools.partial(
    pl.kernel, mesh=_sc_mesh,
    out_type=[jax.ShapeDtypeStruct((_N, _H), jnp.int32)],
    scratch_types=[
        pltpu.VMEM((_CHUNK,), jnp.int32),
        pltpu.VMEM((_CHUNK, _H), jnp.int32),
        pltpu.SemaphoreType.DMA,
    ],
    compiler_params=_sc_params,
)
def _gather_card(idx_c, mem_c, out_c, idx_v, rows_v, sem):
    wid = lax.axis_index("s") * 2 + lax.axis_index("c")
    _gather_tables((idx_c,), (mem_c,), (out_c,), idx_v, rows_v, sem, wid)


_BR = 1000


_PBR = 400


def _txn_body(x_ref, w1_ref, b1_ref, wv1a_ref, bv1_ref, a_ref):
    x = x_ref[...].astype(jnp.bfloat16)
    h = jnp.maximum(
        jnp.dot(x, w1_ref[...], preferred_element_type=jnp.float32) + b1_ref[...],
        0.0).astype(jnp.bfloat16)
    a_ref[...] = (
        jnp.dot(h, wv1a_ref[...], preferred_element_type=jnp.float32)
        + bv1_ref[...])


def _txn_part(txn_x2, W1bd, b1c, Wv1abd, bv1c):
    grid = _N // 2 // _PBR
    return pl.pallas_call(
        _txn_body,
        grid=(grid,),
        in_specs=[
            pl.BlockSpec((_PBR, 2 * _TXN_IN), lambda i: (i, 0)),
            pl.BlockSpec((2 * _TXN_IN, 128), lambda i: (0, 0)),
            pl.BlockSpec((1, 128), lambda i: (0, 0)),
            pl.BlockSpec((128, 128), lambda i: (0, 0)),
            pl.BlockSpec((1, 128), lambda i: (0, 0)),
        ],
        out_specs=pl.BlockSpec((_PBR, 128), lambda i: (i, 0)),
        out_shape=jax.ShapeDtypeStruct((_N // 2, 128), jnp.float32),
        compiler_params=pltpu.CompilerParams(
            dimension_semantics=("arbitrary",),
        ),
    )(txn_x2, W1bd, b1c, Wv1abd, bv1c)


def _unpack_card(g32, pe, po):
    c0, c1 = g32[:, 0:32], g32[:, 32:64]
    c2, c3 = g32[:, 64:96], g32[:, 96:128]
    se = jnp.where(pe == 0, c0, c1)
    so = jnp.where(po == 0, c2, c3)
    hi = jnp.int32(-65536)
    parts = [
        lax.bitcast_convert_type(se << 16, jnp.float32).astype(jnp.bfloat16),
        lax.bitcast_convert_type(se & hi, jnp.float32).astype(jnp.bfloat16),
        lax.bitcast_convert_type(so << 16, jnp.float32).astype(jnp.bfloat16),
        lax.bitcast_convert_type(so & hi, jnp.float32).astype(jnp.bfloat16),
    ]
    return jnp.concatenate(parts, axis=1)


def _tail_body(a_ref, gc_ref, ga_ref, ge_ref, pe_ref, po_ref,
               wv1g_ref, wv2_ref, bv2_ref, out_ref):
    acc = a_ref[...]
    gcard = _unpack_card(jnp.reshape(gc_ref[...], (_PBR, 128)),
                         pe_ref[0], po_ref[0])
    acc += jnp.dot(gcard, wv1g_ref[0:128, :],
                   preferred_element_type=jnp.float32)
    for k, g_ref in enumerate((ga_ref, ge_ref)):
        g = jnp.reshape(g_ref[...], (_PBR, 128)).astype(jnp.bfloat16)
        acc += jnp.dot(g, wv1g_ref[(k + 1) * 128:(k + 2) * 128, :],
                       preferred_element_type=jnp.float32)
    z = jnp.maximum(acc, 0.0)
    out_ref[...] = (
        jnp.dot(z, wv2_ref[...], preferred_element_type=jnp.float32)
        + bv2_ref[...])


def _tail(a2, gc, ga, ge, pe, po, Wv1gbd, Wv2bd, bv2):
    grid = _N // 2 // _PBR
    g_spec = pl.BlockSpec((_PBR * 128,), lambda i: (i,))
    p_spec = pl.BlockSpec((1, _PBR, 1), lambda i: (i, 0, 0))
    return pl.pallas_call(
        _tail_body,
        grid=(grid,),
        in_specs=[
            pl.BlockSpec((_PBR, 128), lambda i: (i, 0)),
            g_spec, g_spec, g_spec,
            p_spec, p_spec,
            pl.BlockSpec((3 * 128, 128), lambda i: (0, 0)),
            pl.BlockSpec((128, 2), lambda i: (0, 0)),
            pl.BlockSpec((1, 1), lambda i: (0, 0)),
        ],
        out_specs=pl.BlockSpec((_PBR, 2), lambda i: (i, 0)),
        out_shape=jax.ShapeDtypeStruct((_N // 2, 2), jnp.float32),
        compiler_params=pltpu.CompilerParams(
            dimension_semantics=("arbitrary",),
        ),
    )(a2, gc, ga, ge, pe, po, Wv1gbd, Wv2bd, bv2)


def _blockdiag(w):
    z = jnp.zeros_like(w)
    return jnp.concatenate(
        [jnp.concatenate([w, z], axis=1), jnp.concatenate([z, w], axis=1)],
        axis=0)


def kernel(txn_x, idx_card, idx_addr, idx_email, mem_card, mem_addr, mem_email,
           W1, b1, unk_card, unk_addr, unk_email, Wv1, bv1, Wv2, bv2):
    wv1b = Wv1.astype(jnp.bfloat16)
    w1b = W1.astype(jnp.bfloat16)
    b1c = jnp.concatenate([b1, b1]).reshape(1, 128)
    bv1c = jnp.concatenate([bv1, bv1]).reshape(1, 128)
    wc = wv1b[_H:2 * _H]
    wcp = jnp.concatenate([wc[0::2], wc[1::2]], axis=0)
    wv1gbd = jnp.concatenate(
        [_blockdiag(wcp)] + [
            _blockdiag(wv1b[(k + 1) * _H:(k + 2) * _H]) for k in (1, 2)
        ], axis=0)
    wv2bd = jnp.concatenate(
        [jnp.concatenate([Wv2, jnp.zeros_like(Wv2)], axis=1),
         jnp.concatenate([jnp.zeros_like(Wv2), Wv2], axis=1)], axis=0)
    a2 = _txn_part(txn_x.reshape(_N // 2, 2 * _TXN_IN),
                   _blockdiag(w1b), b1c, _blockdiag(wv1b[0:_H]), bv1c)
    ic, ia, ie = (i.astype(jnp.int32)
                  for i in (idx_card, idx_addr, idx_email))
    icp = ic.reshape(_N // 2, 2)
    pe = (icp[:, 0] % 2).reshape(_N // 2 // _PBR, _PBR, 1)
    po = (icp[:, 1] % 2).reshape(_N // 2 // _PBR, _PBR, 1)
    pk_card = lax.bitcast_convert_type(
        mem_card.astype(jnp.bfloat16).reshape(mem_card.shape[0] // 2, _H, 2),
        jnp.int32)
    ga, ge = _gather_small(ia, ie, mem_addr, mem_email)
    (gc,) = _gather_card(ic // 2, pk_card)
    out = _tail(a2, gc.reshape(-1), ga.reshape(-1), ge.reshape(-1),
                pe, po, wv1gbd, wv2bd, bv2.reshape(1, 1))
    return out.reshape(_N)


# speedup vs baseline: 24.9901x; 24.9901x over previous
import functools

import jax
import jax.numpy as jnp
from jax import lax
from jax.experimental import pallas as pl
from jax.experimental.pallas import tpu as pltpu
from jax.experimental.pallas import tpu_sc as plsc

_N = 100000
_H = 64
_TXN_IN = 256

_CHUNK = 1024
_SUB = 128
_NCHUNKS = (_N + _CHUNK - 1) // _CHUNK
_LASTBASE = _N - _CHUNK
_NW = 32

_sc_mesh = plsc.VectorSubcoreMesh(core_axis_name="c", subcore_axis_name="s")


def _gather_tables(idx_refs, mem_refs, out_refs, idx_v, rows_v, sem, wid):
    for idx_hbm, mem_hbm, out_hbm in zip(idx_refs, mem_refs, out_refs):
        for j in range((_NCHUNKS + _NW - 1) // _NW):
            c = wid + _NW * j

            @pl.when(c < _NCHUNKS)
            def _():
                base = jnp.minimum(c * _CHUNK, _LASTBASE)
                pltpu.sync_copy(idx_hbm.at[pl.ds(base, _CHUNK)], idx_v)
                cps = [
                    pltpu.async_copy(
                        mem_hbm.at[idx_v.at[pl.ds(q * _SUB, _SUB)]],
                        rows_v.at[pl.ds(q * _SUB, _SUB)],
                        sem,
                    )
                    for q in range(8)
                ]
                for cp in cps:
                    cp.wait()
                pltpu.sync_copy(rows_v, out_hbm.at[pl.ds(base, _CHUNK)])


_sc_scratch = [
    pltpu.VMEM((_CHUNK,), jnp.int32),
    pltpu.VMEM((_CHUNK, _H), jnp.float32),
    pltpu.SemaphoreType.DMA,
]
_sc_params = pltpu.CompilerParams(use_tc_tiling_on_sc=False)


@functools.partial(
    pl.kernel, mesh=_sc_mesh,
    out_type=[jax.ShapeDtypeStruct((_N, _H), jnp.float32)] * 2,
    scratch_types=_sc_scratch, compiler_params=_sc_params,
)
def _gather_small(idx_a, idx_e, mem_a, mem_e, out_a, out_e,
                  idx_v, rows_v, sem):
    wid = lax.axis_index("s") * 2 + lax.axis_index("c")
    _gather_tables((idx_a, idx_e), (mem_a, mem_e), (out_a, out_e),
                   idx_v, rows_v, sem, wid)


@functools.partial(
    pl.kernel, mesh=_sc_mesh,
    out_type=[jax.ShapeDtypeStruct((_N, _H), jnp.float32)],
    scratch_types=_sc_scratch, compiler_params=_sc_params,
)
def _gather_card(idx_c, mem_c, out_c, idx_v, rows_v, sem):
    wid = lax.axis_index("s") * 2 + lax.axis_index("c")
    _gather_tables((idx_c,), (mem_c,), (out_c,), idx_v, rows_v, sem, wid)


_BR = 1000


_PBR = 400


def _txn_body(x_ref, w1_ref, b1_ref, wv1a_ref, bv1_ref, a_ref):
    x = x_ref[...].astype(jnp.bfloat16)
    h = jnp.maximum(
        jnp.dot(x, w1_ref[...], preferred_element_type=jnp.float32) + b1_ref[...],
        0.0).astype(jnp.bfloat16)
    a_ref[...] = (
        jnp.dot(h, wv1a_ref[...], preferred_element_type=jnp.float32)
        + bv1_ref[...])


def _txn_part(txn_x2, W1bd, b1c, Wv1abd, bv1c):
    grid = _N // 2 // _PBR
    return pl.pallas_call(
        _txn_body,
        grid=(grid,),
        in_specs=[
            pl.BlockSpec((_PBR, 2 * _TXN_IN), lambda i: (i, 0)),
            pl.BlockSpec((2 * _TXN_IN, 128), lambda i: (0, 0)),
            pl.BlockSpec((1, 128), lambda i: (0, 0)),
            pl.BlockSpec((128, 128), lambda i: (0, 0)),
            pl.BlockSpec((1, 128), lambda i: (0, 0)),
        ],
        out_specs=pl.BlockSpec((_PBR, 128), lambda i: (i, 0)),
        out_shape=jax.ShapeDtypeStruct((_N // 2, 128), jnp.float32),
        compiler_params=pltpu.CompilerParams(
            dimension_semantics=("arbitrary",),
        ),
    )(txn_x2, W1bd, b1c, Wv1abd, bv1c)


def _tail_body(a_ref, gc_ref, ga_ref, ge_ref,
               wv1g_ref, wv2_ref, bv2_ref, out_ref):
    acc = a_ref[...]
    for k, g_ref in enumerate((gc_ref, ga_ref, ge_ref)):
        g = jnp.reshape(g_ref[...], (_PBR, 128)).astype(jnp.bfloat16)
        acc += jnp.dot(g, wv1g_ref[k * 128:(k + 1) * 128, :],
                       preferred_element_type=jnp.float32)
    z = jnp.maximum(acc, 0.0)
    out_ref[...] = (
        jnp.dot(z, wv2_ref[...], preferred_element_type=jnp.float32)
        + bv2_ref[...])


def _tail(a2, gc, ga, ge, Wv1gbd, Wv2bd, bv2):
    grid = _N // 2 // _PBR
    g_spec = pl.BlockSpec((_PBR * 128,), lambda i: (i,))
    return pl.pallas_call(
        _tail_body,
        grid=(grid,),
        in_specs=[
            pl.BlockSpec((_PBR, 128), lambda i: (i, 0)),
            g_spec, g_spec, g_spec,
            pl.BlockSpec((3 * 128, 128), lambda i: (0, 0)),
            pl.BlockSpec((128, 2), lambda i: (0, 0)),
            pl.BlockSpec((1, 1), lambda i: (0, 0)),
        ],
        out_specs=pl.BlockSpec((_PBR, 2), lambda i: (i, 0)),
        out_shape=jax.ShapeDtypeStruct((_N // 2, 2), jnp.float32),
        compiler_params=pltpu.CompilerParams(
            dimension_semantics=("arbitrary",),
        ),
    )(a2, gc, ga, ge, Wv1gbd, Wv2bd, bv2)


def _blockdiag(w):
    z = jnp.zeros_like(w)
    return jnp.concatenate(
        [jnp.concatenate([w, z], axis=1), jnp.concatenate([z, w], axis=1)],
        axis=0)


def kernel(txn_x, idx_card, idx_addr, idx_email, mem_card, mem_addr, mem_email,
           W1, b1, unk_card, unk_addr, unk_email, Wv1, bv1, Wv2, bv2):
    wv1b = Wv1.astype(jnp.bfloat16)
    w1b = W1.astype(jnp.bfloat16)
    b1c = jnp.concatenate([b1, b1]).reshape(1, 128)
    bv1c = jnp.concatenate([bv1, bv1]).reshape(1, 128)
    wv1gbd = jnp.concatenate(
        [_blockdiag(wv1b[(k + 1) * _H:(k + 2) * _H]) for k in range(3)],
        axis=0)
    wv2bd = jnp.concatenate(
        [jnp.concatenate([Wv2, jnp.zeros_like(Wv2)], axis=1),
         jnp.concatenate([jnp.zeros_like(Wv2), Wv2], axis=1)], axis=0)
    a2 = _txn_part(txn_x.reshape(_N // 2, 2 * _TXN_IN),
                   _blockdiag(w1b), b1c, _blockdiag(wv1b[0:_H]), bv1c)
    ic, ia, ie = (i.astype(jnp.int32)
                  for i in (idx_card, idx_addr, idx_email))
    ga, ge = _gather_small(ia, ie, mem_addr, mem_email)
    (gc,) = _gather_card(ic, mem_card)
    out = _tail(a2, gc.reshape(-1), ga.reshape(-1), ge.reshape(-1),
                wv1gbd, wv2bd, bv2.reshape(1, 1))
    return out.reshape(_N)


# speedup vs baseline: 26.8136x; 1.0730x over previous
import functools

import jax
import jax.numpy as jnp
from jax import lax
from jax.experimental import pallas as pl
from jax.experimental.pallas import tpu as pltpu
from jax.experimental.pallas import tpu_sc as plsc

_N = 100000
_H = 64
_TXN_IN = 256

_CHUNK = 1024
_SUB = 128
_NCHUNKS = (_N + _CHUNK - 1) // _CHUNK
_LASTBASE = _N - _CHUNK
_NW = 32

_sc_mesh = plsc.VectorSubcoreMesh(core_axis_name="c", subcore_axis_name="s")


def _gather_tables(idx_refs, mem_refs, out_refs, idx_v, rows_v, sem, wid):
    for idx_hbm, mem_hbm, out_hbm in zip(idx_refs, mem_refs, out_refs):
        for j in range((_NCHUNKS + _NW - 1) // _NW):
            c = wid + _NW * j

            @pl.when(c < _NCHUNKS)
            def _():
                base = jnp.minimum(c * _CHUNK, _LASTBASE)
                pltpu.sync_copy(idx_hbm.at[pl.ds(base, _CHUNK)], idx_v)
                cps = [
                    pltpu.async_copy(
                        mem_hbm.at[idx_v.at[pl.ds(q * _SUB, _SUB)]],
                        rows_v.at[pl.ds(q * _SUB, _SUB)],
                        sem,
                    )
                    for q in range(8)
                ]
                for cp in cps:
                    cp.wait()
                pltpu.sync_copy(rows_v, out_hbm.at[pl.ds(base, _CHUNK)])


_sc_scratch = [
    pltpu.VMEM((_CHUNK,), jnp.int32),
    pltpu.VMEM((_CHUNK, _H), jnp.float32),
    pltpu.SemaphoreType.DMA,
]
_sc_params = pltpu.CompilerParams(use_tc_tiling_on_sc=False)


@functools.partial(
    pl.kernel, mesh=_sc_mesh,
    out_type=[jax.ShapeDtypeStruct((_N, _H), jnp.float32)] * 2,
    scratch_types=_sc_scratch, compiler_params=_sc_params,
)
def _gather_small(idx_a, idx_e, mem_a, mem_e, out_a, out_e,
                  idx_v, rows_v, sem):
    wid = lax.axis_index("s") * 2 + lax.axis_index("c")
    _gather_tables((idx_a, idx_e), (mem_a, mem_e), (out_a, out_e),
                   idx_v, rows_v, sem, wid)


@functools.partial(
    pl.kernel, mesh=_sc_mesh,
    out_type=[jax.ShapeDtypeStruct((_N, _H), jnp.float32)],
    scratch_types=_sc_scratch, compiler_params=_sc_params,
)
def _gather_card(idx_c, mem_c, out_c, idx_v, rows_v, sem):
    wid = lax.axis_index("s") * 2 + lax.axis_index("c")
    _gather_tables((idx_c,), (mem_c,), (out_c,), idx_v, rows_v, sem, wid)


_BR = 1000


_PBR = 400


def _tail_body(x_ref, gc_ref, ga_ref, ge_ref,
               w1_ref, b1_ref, wv1a_ref, bv1_ref,
               wv1g_ref, wv2_ref, bv2_ref, out_ref):
    x = x_ref[...].astype(jnp.bfloat16)
    h = jnp.maximum(
        jnp.dot(x, w1_ref[...], preferred_element_type=jnp.float32) + b1_ref[...],
        0.0).astype(jnp.bfloat16)
    acc = (jnp.dot(h, wv1a_ref[...], preferred_element_type=jnp.float32)
           + bv1_ref[...])
    for k, g_ref in enumerate((gc_ref, ga_ref, ge_ref)):
        g = jnp.reshape(g_ref[...], (_PBR, 128)).astype(jnp.bfloat16)
        acc += jnp.dot(g, wv1g_ref[k * 128:(k + 1) * 128, :],
                       preferred_element_type=jnp.float32)
    z = jnp.maximum(acc, 0.0)
    out_ref[...] = (
        jnp.dot(z, wv2_ref[...], preferred_element_type=jnp.float32)
        + bv2_ref[...])


def _tail(x2, gc, ga, ge, W1bd, b1c, Wv1abd, bv1c, Wv1gbd, Wv2bd, bv2):
    grid = _N // 2 // _PBR
    g_spec = pl.BlockSpec((_PBR * 128,), lambda i: (i,))
    return pl.pallas_call(
        _tail_body,
        grid=(grid,),
        in_specs=[
            pl.BlockSpec((_PBR, 2 * _TXN_IN), lambda i: (i, 0)),
            g_spec, g_spec, g_spec,
            pl.BlockSpec((2 * _TXN_IN, 128), lambda i: (0, 0)),
            pl.BlockSpec((1, 128), lambda i: (0, 0)),
            pl.BlockSpec((128, 128), lambda i: (0, 0)),
            pl.BlockSpec((1, 128), lambda i: (0, 0)),
            pl.BlockSpec((3 * 128, 128), lambda i: (0, 0)),
            pl.BlockSpec((128, 2), lambda i: (0, 0)),
            pl.BlockSpec((1, 1), lambda i: (0, 0)),
        ],
        out_specs=pl.BlockSpec((_PBR, 2), lambda i: (i, 0)),
        out_shape=jax.ShapeDtypeStruct((_N // 2, 2), jnp.float32),
        compiler_params=pltpu.CompilerParams(
            dimension_semantics=("arbitrary",),
        ),
    )(x2, gc, ga, ge, W1bd, b1c, Wv1abd, bv1c, Wv1gbd, Wv2bd, bv2)


def _blockdiag(w):
    z = jnp.zeros_like(w)
    return jnp.concatenate(
        [jnp.concatenate([w, z], axis=1), jnp.concatenate([z, w], axis=1)],
        axis=0)


def kernel(txn_x, idx_card, idx_addr, idx_email, mem_card, mem_addr, mem_email,
           W1, b1, unk_card, unk_addr, unk_email, Wv1, bv1, Wv2, bv2):
    wv1b = Wv1.astype(jnp.bfloat16)
    w1b = W1.astype(jnp.bfloat16)
    b1c = jnp.concatenate([b1, b1]).reshape(1, 128)
    bv1c = jnp.concatenate([bv1, bv1]).reshape(1, 128)
    wv1gbd = jnp.concatenate(
        [_blockdiag(wv1b[(k + 1) * _H:(k + 2) * _H]) for k in range(3)],
        axis=0)
    wv2bd = jnp.concatenate(
        [jnp.concatenate([Wv2, jnp.zeros_like(Wv2)], axis=1),
         jnp.concatenate([jnp.zeros_like(Wv2), Wv2], axis=1)], axis=0)
    ic, ia, ie = (i.astype(jnp.int32)
                  for i in (idx_card, idx_addr, idx_email))
    ga, ge = _gather_small(ia, ie, mem_addr, mem_email)
    (gc,) = _gather_card(ic, mem_card)
    out = _tail(txn_x.reshape(_N // 2, 2 * _TXN_IN),
                gc.reshape(-1), ga.reshape(-1), ge.reshape(-1),
                _blockdiag(w1b), b1c, _blockdiag(wv1b[0:_H]), bv1c,
                wv1gbd, wv2bd, bv2.reshape(1, 1))
    return out.reshape(_N)
